# baseline (speedup 1.0000x reference)
import functools

import jax
import jax.numpy as jnp
from jax.experimental import pallas as pl

B, M, D = 4096, 6, 128
E, TOPK, FUT = 6, 2, 60
N = B * M
OUT = FUT * 2


def _gelu(v):
    return v * 0.5 * (1.0 + jax.lax.erf(v * 0.7071067811865476))


_ERF_C = (1.128366727910593, -0.3758777741346081, 0.11201253111177815,
          -0.02579328726893187, 0.004504954585628093,
          -0.0005704712144833727, 4.849776727069829e-05,
          -2.4370926315406327e-06, 5.412213267639847e-08)


def _gelu_fast(v):
    z = jnp.clip(v * 0.7071067811865476, -3.0, 3.0)
    u = z * z
    p = jnp.float32(_ERF_C[8])
    for c in _ERF_C[7::-1]:
        p = p * u + jnp.float32(c)
    return 0.5 * v + (0.5 * v) * (z * p)


def _fused_body(x_ref, r_w1, r_b1, r_w2, r_b2, r_w3, r_b3,
                t_w1, t_b1, t_w2, t_b2, t_w3, t_b3,
                s_w1, s_b1, s_w2, s_b2, s_w3, s_b3,
                traj_ref, score_ref, probs_ref, psum_ref):
    x = x_ref[...]
    f32 = jnp.float32
    dot = functools.partial(jnp.dot, preferred_element_type=f32)

    h = _gelu(dot(x, r_w1[...]) + r_b1[...])
    h = _gelu(dot(h, r_w2[...]) + r_b2[...])
    logits = dot(h, r_w3[...]) + r_b3[...]

    m = jnp.max(logits, axis=-1, keepdims=True)
    ex = jnp.exp(logits - m)
    denom = jnp.sum(ex, axis=-1, keepdims=True)
    probs = ex / denom
    probs_ref[...] = probs
    psum_ref[...] = jnp.sum(probs, axis=0).reshape(1, 1, E)

    T = x.shape[0]
    col = jax.lax.broadcasted_iota(jnp.int32, (T, E), 1)
    m0 = jnp.max(logits, axis=-1)
    i0 = jnp.min(jnp.where(logits == m0[:, None], col, E), axis=-1)
    masked = jnp.where(col == i0[:, None], -jnp.inf, logits)
    m1 = jnp.max(masked, axis=-1)
    i1 = jnp.min(jnp.where(masked == m1[:, None], col, E), axis=-1)
    e1 = jnp.exp(m1 - m0)
    p0 = 1.0 / (1.0 + e1)
    p1 = e1 / (1.0 + e1)

    bf = jnp.bfloat16
    xb = x.astype(bf)
    acc_t = jnp.zeros((T, OUT), f32)
    acc_s = jnp.zeros((T,), f32)
    for e in range(E):
        th = _gelu_fast(dot(xb, t_w1[e]) + t_b1[e])
        th = _gelu_fast(dot(th.astype(bf), t_w2[e]) + t_b2[e])
        tr = dot(th.astype(bf), t_w3[e]) + t_b3[e]
        sh = _gelu_fast(dot(xb, s_w1[e]) + s_b1[e])
        sh = _gelu_fast(dot(sh.astype(bf), s_w2[e]) + s_b2[e])
        sc = jnp.sum(sh * s_w3[e, :, 0].astype(f32), axis=-1) + s_b3[e, 0]
        w = jnp.where(i0 == e, p0, 0.0) + jnp.where(i1 == e, p1, 0.0)
        acc_t = acc_t + w[:, None] * tr
        acc_s = acc_s + w * sc
    traj_ref[...] = acc_t
    score_ref[...] = acc_s[:, None]


def kernel(mode_features, r_w1, r_b1, r_w2, r_b2, r_w3, r_b3,
           t_w1, t_b1, t_w2, t_b2, t_w3, t_b3,
           s_w1, s_b1, s_w2, s_b2, s_w3, s_b3):
    x = mode_features.reshape(N, D)
    r_b1 = r_b1.reshape(1, -1)
    r_b2 = r_b2.reshape(1, -1)
    r_b3 = r_b3.reshape(1, -1)
    bf = jnp.bfloat16
    t_w1 = t_w1.astype(bf)
    t_w2 = t_w2.astype(bf)
    t_w3 = t_w3.astype(bf)
    s_w1 = s_w1.astype(bf)
    s_w2 = s_w2.astype(bf)
    s_w3 = s_w3.astype(bf)

    TILE = 1024
    grid = (N // TILE,)

    def tok_map(i):
        return (i, 0)

    def const_map2(i):
        return (0, 0)

    def const_map3(i):
        return (0, 0, 0)

    full2 = lambda a: pl.BlockSpec(a.shape, const_map2)
    full3 = lambda a: pl.BlockSpec(a.shape, const_map3)

    traj, score, probs, psum = pl.pallas_call(
        _fused_body,
        grid=grid,
        in_specs=[
            pl.BlockSpec((TILE, D), tok_map),
            full2(r_w1), full2(r_b1), full2(r_w2), full2(r_b2),
            full2(r_w3), full2(r_b3),
            full3(t_w1), full2(t_b1), full3(t_w2), full2(t_b2),
            full3(t_w3), full2(t_b3),
            full3(s_w1), full2(s_b1), full3(s_w2), full2(s_b2),
            full3(s_w3), full2(s_b3),
        ],
        out_specs=[
            pl.BlockSpec((TILE, OUT), tok_map),
            pl.BlockSpec((TILE, 1), tok_map),
            pl.BlockSpec((TILE, E), tok_map),
            pl.BlockSpec((1, 1, E), lambda i: (i, 0, 0)),
        ],
        out_shape=[
            jax.ShapeDtypeStruct((N, OUT), jnp.float32),
            jax.ShapeDtypeStruct((N, 1), jnp.float32),
            jax.ShapeDtypeStruct((N, E), jnp.float32),
            jax.ShapeDtypeStruct((grid[0], 1, E), jnp.float32),
        ],
    )(x, r_w1, r_b1, r_w2, r_b2, r_w3, r_b3,
      t_w1, t_b1, t_w2, t_b2, t_w3, t_b3,
      s_w1, s_b1, s_w2, s_b2, s_w3, s_b3)

    trajectories = traj.reshape(B, M, FUT, 2)
    scores = score.reshape(B, M)
    probs_out = probs.reshape(B, M, E)
    avg = psum.reshape(-1, E).sum(axis=0) / N
    entropy = -(avg * jnp.log(avg + 1e-08)).sum()
    load_balance_loss = -entropy * 0.01
    uniform = jnp.ones_like(avg) / E
    l2_loss = jnp.mean((avg - uniform) ** 2)
    aux_loss = load_balance_loss + 0.01 * l2_loss
    return (trajectories, scores, aux_loss, probs_out)


# speedup vs baseline: 4.1860x; 4.1860x over previous
import functools

import jax
import jax.numpy as jnp
from jax.experimental import pallas as pl

B, M, D = 4096, 6, 128
E, TOPK, FUT = 6, 2, 60
N = B * M
OUT = FUT * 2


def _gelu(v):
    return v * 0.5 * (1.0 + jax.lax.erf(v * 0.7071067811865476))


def _fused_body(x_ref, r_w1, r_b1, r_w2, r_b2, r_w3, r_b3,
                t_w1, t_b1, t_w2, t_b2, t_w3, t_b3,
                s_w1, s_b1, s_w2, s_b2, s_w3, s_b3,
                traj_ref, score_ref, probs_ref, psum_ref):
    x = x_ref[...]
    f32 = jnp.float32
    dot = functools.partial(jnp.dot, preferred_element_type=f32)

    h = _gelu(dot(x, r_w1[...]) + r_b1[...])
    h = _gelu(dot(h, r_w2[...]) + r_b2[...])
    logits = dot(h, r_w3[...]) + r_b3[...]

    m = jnp.max(logits, axis=-1, keepdims=True)
    ex = jnp.exp(logits - m)
    denom = jnp.sum(ex, axis=-1, keepdims=True)
    probs = ex / denom
    probs_ref[...] = probs
    psum_ref[...] = jnp.sum(probs, axis=0).reshape(1, 1, E)

    T = x.shape[0]
    col = jax.lax.broadcasted_iota(jnp.int32, (T, E), 1)
    m0 = jnp.max(logits, axis=-1)
    i0 = jnp.min(jnp.where(logits == m0[:, None], col, E), axis=-1)
    masked = jnp.where(col == i0[:, None], -jnp.inf, logits)
    m1 = jnp.max(masked, axis=-1)
    i1 = jnp.min(jnp.where(masked == m1[:, None], col, E), axis=-1)
    e1 = jnp.exp(m1 - m0)
    p0 = 1.0 / (1.0 + e1)
    p1 = e1 / (1.0 + e1)

    bf = jnp.bfloat16
    xb = x.astype(bf)
    acc_t = jnp.zeros((T, OUT), f32)
    acc_s = jnp.zeros((T,), f32)
    for e in range(E):
        th = _gelu(dot(xb, t_w1[e]) + t_b1[e])
        th = _gelu(dot(th.astype(bf), t_w2[e]) + t_b2[e])
        tr = dot(th.astype(bf), t_w3[e]) + t_b3[e]
        sh = _gelu(dot(xb, s_w1[e]) + s_b1[e])
        sh = _gelu(dot(sh.astype(bf), s_w2[e]) + s_b2[e])
        sc = jnp.sum(sh * s_w3[e, :, 0].astype(f32), axis=-1) + s_b3[e, 0]
        w = jnp.where(i0 == e, p0, 0.0) + jnp.where(i1 == e, p1, 0.0)
        acc_t = acc_t + w[:, None] * tr
        acc_s = acc_s + w * sc
    traj_ref[...] = jnp.transpose(acc_t)
    score_ref[...] = acc_s[:, None]


def kernel(mode_features, r_w1, r_b1, r_w2, r_b2, r_w3, r_b3,
           t_w1, t_b1, t_w2, t_b2, t_w3, t_b3,
           s_w1, s_b1, s_w2, s_b2, s_w3, s_b3):
    x = jnp.transpose(mode_features, (1, 0, 2)).reshape(N, D)
    r_b1 = r_b1.reshape(1, -1)
    r_b2 = r_b2.reshape(1, -1)
    r_b3 = r_b3.reshape(1, -1)
    bf = jnp.bfloat16
    t_w1 = t_w1.astype(bf)
    t_w2 = t_w2.astype(bf)
    t_w3 = t_w3.astype(bf)
    s_w1 = s_w1.astype(bf)
    s_w2 = s_w2.astype(bf)
    s_w3 = s_w3.astype(bf)

    TILE = 1024
    PERM = B // TILE
    grid = (N // TILE,)

    def tok_map(i):
        return (i, 0)

    def const_map2(i):
        return (0, 0)

    def const_map3(i):
        return (0, 0, 0)

    full2 = lambda a: pl.BlockSpec(a.shape, const_map2)
    full3 = lambda a: pl.BlockSpec(a.shape, const_map3)

    traj, score, probs, psum = pl.pallas_call(
        _fused_body,
        grid=grid,
        in_specs=[
            pl.BlockSpec((TILE, D), tok_map),
            full2(r_w1), full2(r_b1), full2(r_w2), full2(r_b2),
            full2(r_w3), full2(r_b3),
            full3(t_w1), full2(t_b1), full3(t_w2), full2(t_b2),
            full3(t_w3), full2(t_b3),
            full3(s_w1), full2(s_b1), full3(s_w2), full2(s_b2),
            full3(s_w3), full2(s_b3),
        ],
        out_specs=[
            pl.BlockSpec((OUT, TILE), lambda i: (i // PERM, i % PERM)),
            pl.BlockSpec((TILE, 1), tok_map),
            pl.BlockSpec((TILE, E), tok_map),
            pl.BlockSpec((1, 1, E), lambda i: (i, 0, 0)),
        ],
        out_shape=[
            jax.ShapeDtypeStruct((M * OUT, B), jnp.float32),
            jax.ShapeDtypeStruct((N, 1), jnp.float32),
            jax.ShapeDtypeStruct((N, E), jnp.float32),
            jax.ShapeDtypeStruct((grid[0], 1, E), jnp.float32),
        ],
    )(x, r_w1, r_b1, r_w2, r_b2, r_w3, r_b3,
      t_w1, t_b1, t_w2, t_b2, t_w3, t_b3,
      s_w1, s_b1, s_w2, s_b2, s_w3, s_b3)

    trajectories = traj.reshape(M, FUT, 2, B).transpose(3, 0, 1, 2)
    scores = score.reshape(M, B).transpose(1, 0)
    probs_out = probs.reshape(M, B, E).transpose(1, 0, 2)
    avg = psum.reshape(-1, E).sum(axis=0) / N
    entropy = -(avg * jnp.log(avg + 1e-08)).sum()
    load_balance_loss = -entropy * 0.01
    uniform = jnp.ones_like(avg) / E
    l2_loss = jnp.mean((avg - uniform) ** 2)
    aux_loss = load_balance_loss + 0.01 * l2_loss
    return (trajectories, scores, aux_loss, probs_out)


# speedup vs baseline: 4.2282x; 1.0101x over previous
import functools

import jax
import jax.numpy as jnp
from jax.experimental import pallas as pl

B, M, D = 4096, 6, 128
E, TOPK, FUT = 6, 2, 60
N = B * M
OUT = FUT * 2


def _gelu(v):
    return v * 0.5 * (1.0 + jax.lax.erf(v * 0.7071067811865476))


def _fused_body(x_ref, r_w1, r_b1, r_w2, r_b2, r_w3, r_b3,
                t_w1, t_b1, t_w2, t_b2, t_w3, t_b3,
                s_w1, s_b1, s_w2, s_b2, s_w3, s_b3,
                traj_ref, score_ref, probs_ref, psum_ref):
    x = x_ref[...]
    f32 = jnp.float32
    dot = functools.partial(jnp.dot, preferred_element_type=f32)

    h = _gelu(dot(x, r_w1[...]) + r_b1[...])
    h = _gelu(dot(h, r_w2[...]) + r_b2[...])
    logits = dot(h, r_w3[...]) + r_b3[...]

    m = jnp.max(logits, axis=-1, keepdims=True)
    ex = jnp.exp(logits - m)
    denom = jnp.sum(ex, axis=-1, keepdims=True)
    probs = ex / denom
    probs_ref[...] = probs
    psum_ref[...] = jnp.sum(probs, axis=0).reshape(1, 1, E)

    T = x.shape[0]
    col = jax.lax.broadcasted_iota(jnp.int32, (T, E), 1)
    m0 = jnp.max(logits, axis=-1)
    i0 = jnp.min(jnp.where(logits == m0[:, None], col, E), axis=-1)
    masked = jnp.where(col == i0[:, None], -jnp.inf, logits)
    m1 = jnp.max(masked, axis=-1)
    i1 = jnp.min(jnp.where(masked == m1[:, None], col, E), axis=-1)
    e1 = jnp.exp(m1 - m0)
    p0 = 1.0 / (1.0 + e1)
    p1 = e1 / (1.0 + e1)

    bf = jnp.bfloat16
    xb = x.astype(bf)
    acc_t = jnp.zeros((T, OUT), f32)
    acc_s = jnp.zeros((T,), f32)
    for e in range(E):
        th = _gelu(dot(xb, t_w1[e]) + t_b1[e])
        th = _gelu(dot(th.astype(bf), t_w2[e]) + t_b2[e])
        tr = dot(th.astype(bf), t_w3[e]) + t_b3[e]
        sh = _gelu(dot(xb, s_w1[e]) + s_b1[e])
        sh = _gelu(dot(sh.astype(bf), s_w2[e]) + s_b2[e])
        sc = jnp.sum(sh * s_w3[e, :, 0].astype(f32), axis=-1) + s_b3[e, 0]
        w = jnp.where(i0 == e, p0, 0.0) + jnp.where(i1 == e, p1, 0.0)
        acc_t = acc_t + w[:, None] * tr
        acc_s = acc_s + w * sc
    traj_ref[...] = jnp.transpose(acc_t)
    score_ref[...] = acc_s[:, None]


def kernel(mode_features, r_w1, r_b1, r_w2, r_b2, r_w3, r_b3,
           t_w1, t_b1, t_w2, t_b2, t_w3, t_b3,
           s_w1, s_b1, s_w2, s_b2, s_w3, s_b3):
    x = jnp.transpose(mode_features, (1, 0, 2)).reshape(N, D)
    r_b1 = r_b1.reshape(1, -1)
    r_b2 = r_b2.reshape(1, -1)
    r_b3 = r_b3.reshape(1, -1)
    bf = jnp.bfloat16
    t_w1 = t_w1.astype(bf)
    t_w2 = t_w2.astype(bf)
    t_w3 = t_w3.astype(bf)
    s_w1 = s_w1.astype(bf)
    s_w2 = s_w2.astype(bf)
    s_w3 = s_w3.astype(bf)

    TILE = 2048
    PERM = B // TILE
    grid = (N // TILE,)

    def tok_map(i):
        return (i, 0)

    def const_map2(i):
        return (0, 0)

    def const_map3(i):
        return (0, 0, 0)

    full2 = lambda a: pl.BlockSpec(a.shape, const_map2)
    full3 = lambda a: pl.BlockSpec(a.shape, const_map3)

    traj, score, probs, psum = pl.pallas_call(
        _fused_body,
        grid=grid,
        in_specs=[
            pl.BlockSpec((TILE, D), tok_map),
            full2(r_w1), full2(r_b1), full2(r_w2), full2(r_b2),
            full2(r_w3), full2(r_b3),
            full3(t_w1), full2(t_b1), full3(t_w2), full2(t_b2),
            full3(t_w3), full2(t_b3),
            full3(s_w1), full2(s_b1), full3(s_w2), full2(s_b2),
            full3(s_w3), full2(s_b3),
        ],
        out_specs=[
            pl.BlockSpec((OUT, TILE), lambda i: (i // PERM, i % PERM)),
            pl.BlockSpec((TILE, 1), tok_map),
            pl.BlockSpec((TILE, E), tok_map),
            pl.BlockSpec((1, 1, E), lambda i: (i, 0, 0)),
        ],
        out_shape=[
            jax.ShapeDtypeStruct((M * OUT, B), jnp.float32),
            jax.ShapeDtypeStruct((N, 1), jnp.float32),
            jax.ShapeDtypeStruct((N, E), jnp.float32),
            jax.ShapeDtypeStruct((grid[0], 1, E), jnp.float32),
        ],
    )(x, r_w1, r_b1, r_w2, r_b2, r_w3, r_b3,
      t_w1, t_b1, t_w2, t_b2, t_w3, t_b3,
      s_w1, s_b1, s_w2, s_b2, s_w3, s_b3)

    trajectories = traj.reshape(M, FUT, 2, B).transpose(3, 0, 1, 2)
    scores = score.reshape(M, B).transpose(1, 0)
    probs_out = probs.reshape(M, B, E).transpose(1, 0, 2)
    avg = psum.reshape(-1, E).sum(axis=0) / N
    entropy = -(avg * jnp.log(avg + 1e-08)).sum()
    load_balance_loss = -entropy * 0.01
    uniform = jnp.ones_like(avg) / E
    l2_loss = jnp.mean((avg - uniform) ** 2)
    aux_loss = load_balance_loss + 0.01 * l2_loss
    return (trajectories, scores, aux_loss, probs_out)


# speedup vs baseline: 5.0056x; 1.1839x over previous
import functools

import jax
import jax.numpy as jnp
from jax.experimental import pallas as pl

B, M, D = 4096, 6, 128
E, TOPK, FUT = 6, 2, 60
N = B * M
OUT = FUT * 2


def _gelu2(v):
    return v + v * jax.lax.erf(v * 0.7071067811865476)


def _fused_body(x_ref, r_w1, r_b1, r_w2, r_b2, r_w3, r_b3,
                t_w1, t_b1, t_w2, t_b2, t_w3, t_b3,
                s_w1, s_b1, s_w2, s_b2, s_w3, s_b3,
                traj_ref, score_ref, probs_ref, psum_ref):
    f32 = jnp.float32
    dot = functools.partial(jnp.dot, preferred_element_type=f32)
    xt = jnp.transpose(x_ref[...])

    h = _gelu2(dot(r_w1[...], xt) + r_b1[...])
    h = _gelu2(dot(r_w2[...], h) + r_b2[...])
    logits = dot(r_w3[...], h) + r_b3[...]

    mx = jnp.max(logits, axis=0, keepdims=True)
    ex = jnp.exp(logits - mx)
    den = jnp.sum(ex, axis=0, keepdims=True)
    probs = ex / den
    T = xt.shape[1]
    probs_ref[...] = probs.reshape(1, E, T)
    psum_ref[...] = jnp.sum(probs, axis=1).reshape(1, 1, E)

    row = jax.lax.broadcasted_iota(jnp.int32, (E, T), 0)
    m0 = jnp.max(logits, axis=0)
    i0 = jnp.min(jnp.where(logits == m0[None, :], row, E), axis=0)
    masked = jnp.where(row == i0[None, :], -jnp.inf, logits)
    m1 = jnp.max(masked, axis=0)
    i1 = jnp.min(jnp.where(masked == m1[None, :], row, E), axis=0)
    e1 = jnp.exp(m1 - m0)
    p0 = 1.0 / (1.0 + e1)
    p1 = e1 / (1.0 + e1)
    wfull = (jnp.where(row == i0[None, :], p0[None, :], 0.0)
             + jnp.where(row == i1[None, :], p1[None, :], 0.0))

    bf = jnp.bfloat16
    xb = xt.astype(bf)
    acc_t = jnp.zeros((OUT, T), f32)
    acc_s = jnp.zeros((T,), f32)
    for e in range(E):
        th = _gelu2(dot(t_w1[e], xb) + t_b1[e])
        th = _gelu2(dot(t_w2[e], th.astype(bf)) + t_b2[e])
        tr = dot(t_w3[e], th.astype(bf)) + t_b3[e]
        sh = _gelu2(dot(s_w1[e], xb) + s_b1[e])
        sh = _gelu2(dot(s_w2[e], sh.astype(bf)) + s_b2[e])
        sc = jnp.sum(sh * s_w3[e], axis=0) + s_b3[e, 0]
        we = wfull[e:e + 1, :]
        acc_t = acc_t + we * tr
        acc_s = acc_s + wfull[e] * sc
    traj_ref[...] = acc_t
    score_ref[...] = acc_s.reshape(1, 1, T)


def kernel(mode_features, r_w1, r_b1, r_w2, r_b2, r_w3, r_b3,
           t_w1, t_b1, t_w2, t_b2, t_w3, t_b3,
           s_w1, s_b1, s_w2, s_b2, s_w3, s_b3):
    x = jnp.transpose(mode_features, (1, 0, 2)).reshape(N, D)
    bf = jnp.bfloat16
    r_w1 = r_w1.T
    r_b1 = r_b1.reshape(-1, 1)
    r_w2 = (r_w2 * 0.5).T
    r_b2 = r_b2.reshape(-1, 1)
    r_w3 = (r_w3 * 0.5).T
    r_b3 = r_b3.reshape(-1, 1)
    t_w1 = t_w1.transpose(0, 2, 1).astype(bf)
    t_b1 = t_b1.reshape(E, -1, 1)
    t_w2 = (t_w2 * 0.5).transpose(0, 2, 1).astype(bf)
    t_b2 = t_b2.reshape(E, -1, 1)
    t_w3 = (t_w3 * 0.5).transpose(0, 2, 1).astype(bf)
    t_b3 = t_b3.reshape(E, -1, 1)
    s_w1 = s_w1.transpose(0, 2, 1).astype(bf)
    s_b1 = s_b1.reshape(E, -1, 1)
    s_w2 = (s_w2 * 0.5).transpose(0, 2, 1).astype(bf)
    s_b2 = s_b2.reshape(E, -1, 1)
    s_w3 = s_w3 * 0.5

    TILE = 2048
    PERM = B // TILE
    grid = (N // TILE,)

    def tok_map(i):
        return (i, 0)

    def mmaj_map(i):
        return (i // PERM, i % PERM)

    def const_map2(i):
        return (0, 0)

    def const_map3(i):
        return (0, 0, 0)

    full2 = lambda a: pl.BlockSpec(a.shape, const_map2)
    full3 = lambda a: pl.BlockSpec(a.shape, const_map3)

    traj, score, probs, psum = pl.pallas_call(
        _fused_body,
        grid=grid,
        in_specs=[
            pl.BlockSpec((TILE, D), tok_map),
            full2(r_w1), full2(r_b1), full2(r_w2), full2(r_b2),
            full2(r_w3), full2(r_b3),
            full3(t_w1), full3(t_b1), full3(t_w2), full3(t_b2),
            full3(t_w3), full3(t_b3),
            full3(s_w1), full3(s_b1), full3(s_w2), full3(s_b2),
            full3(s_w3), full2(s_b3),
        ],
        out_specs=[
            pl.BlockSpec((OUT, TILE), mmaj_map),
            pl.BlockSpec((1, 1, TILE), lambda i: (i // PERM, 0, i % PERM)),
            pl.BlockSpec((1, E, TILE), lambda i: (i // PERM, 0, i % PERM)),
            pl.BlockSpec((1, 1, E), lambda i: (i, 0, 0)),
        ],
        out_shape=[
            jax.ShapeDtypeStruct((M * OUT, B), jnp.float32),
            jax.ShapeDtypeStruct((M, 1, B), jnp.float32),
            jax.ShapeDtypeStruct((M, E, B), jnp.float32),
            jax.ShapeDtypeStruct((grid[0], 1, E), jnp.float32),
        ],
    )(x, r_w1, r_b1, r_w2, r_b2, r_w3, r_b3,
      t_w1, t_b1, t_w2, t_b2, t_w3, t_b3,
      s_w1, s_b1, s_w2, s_b2, s_w3, s_b3)

    trajectories = traj.reshape(M, FUT, 2, B).transpose(3, 0, 1, 2)
    scores = score.reshape(M, B).transpose(1, 0)
    probs_out = probs.transpose(2, 0, 1)
    avg = psum.reshape(-1, E).sum(axis=0) / N
    entropy = -(avg * jnp.log(avg + 1e-08)).sum()
    load_balance_loss = -entropy * 0.01
    uniform = jnp.ones_like(avg) / E
    l2_loss = jnp.mean((avg - uniform) ** 2)
    aux_loss = load_balance_loss + 0.01 * l2_loss
    return (trajectories, scores, aux_loss, probs_out)


# speedup vs baseline: 5.1856x; 1.0360x over previous
import functools

import jax
import jax.numpy as jnp
from jax.experimental import pallas as pl

B, M, D = 4096, 6, 128
E, TOPK, FUT = 6, 2, 60
N = B * M
OUT = FUT * 2


def _gelu2(v):
    return v + v * jax.lax.erf(v * 0.7071067811865476)


def _fused_body(x_ref, r_w1, r_b1, r_w2, r_b2, r_w3, r_b3,
                t_w1, t_b1, t_w2, t_b2, t_w3, t_b3,
                s_w1, s_b1, s_w2, s_b2, s_w3, s_b3,
                traj_ref, score_ref, probs_ref, psum_ref):
    f32 = jnp.float32
    dot = functools.partial(jnp.dot, preferred_element_type=f32)
    xt = jnp.transpose(x_ref[...])

    h = _gelu2(dot(r_w1[...], xt) + r_b1[...])
    h = _gelu2(dot(r_w2[...], h) + r_b2[...])
    logits = dot(r_w3[...], h) + r_b3[...]

    mx = jnp.max(logits, axis=0, keepdims=True)
    ex = jnp.exp(logits - mx)
    den = jnp.sum(ex, axis=0, keepdims=True)
    probs = ex / den
    T = xt.shape[1]
    probs_ref[...] = probs.reshape(1, E, T)
    psum_ref[...] = jnp.sum(probs, axis=1).reshape(1, 1, E)

    row = jax.lax.broadcasted_iota(jnp.int32, (E, T), 0)
    m0 = jnp.max(logits, axis=0)
    i0 = jnp.min(jnp.where(logits == m0[None, :], row, E), axis=0)
    masked = jnp.where(row == i0[None, :], -jnp.inf, logits)
    m1 = jnp.max(masked, axis=0)
    i1 = jnp.min(jnp.where(masked == m1[None, :], row, E), axis=0)
    e1 = jnp.exp(m1 - m0)
    p0 = 1.0 / (1.0 + e1)
    p1 = e1 / (1.0 + e1)
    wfull = (jnp.where(row == i0[None, :], p0[None, :], 0.0)
             + jnp.where(row == i1[None, :], p1[None, :], 0.0))

    bf = jnp.bfloat16
    xb = xt.astype(bf)
    acc_t = jnp.zeros((OUT, T), f32)
    acc_s = jnp.zeros((T,), f32)
    for e in range(E):
        th = _gelu2(dot(t_w1[e], xb) + t_b1[e])
        th = _gelu2(dot(t_w2[e], th.astype(bf)) + t_b2[e])
        tr = dot(t_w3[e], th.astype(bf)) + t_b3[e]
        sh = _gelu2(dot(s_w1[e], xb) + s_b1[e])
        sh = _gelu2(dot(s_w2[e], sh.astype(bf)) + s_b2[e])
        sc = jnp.sum(sh * s_w3[e], axis=0) + s_b3[e, 0]
        we = wfull[e:e + 1, :]
        acc_t = acc_t + we * tr
        acc_s = acc_s + wfull[e] * sc
    traj_ref[...] = acc_t
    score_ref[...] = acc_s.reshape(1, 1, T)


def kernel(mode_features, r_w1, r_b1, r_w2, r_b2, r_w3, r_b3,
           t_w1, t_b1, t_w2, t_b2, t_w3, t_b3,
           s_w1, s_b1, s_w2, s_b2, s_w3, s_b3):
    x = jnp.transpose(mode_features, (1, 0, 2)).reshape(N, D)
    bf = jnp.bfloat16
    r_w1 = r_w1.T
    r_b1 = r_b1.reshape(-1, 1)
    r_w2 = (r_w2 * 0.5).T
    r_b2 = r_b2.reshape(-1, 1)
    r_w3 = (r_w3 * 0.5).T
    r_b3 = r_b3.reshape(-1, 1)
    t_w1 = t_w1.transpose(0, 2, 1).astype(bf)
    t_b1 = t_b1.reshape(E, -1, 1)
    t_w2 = (t_w2 * 0.5).transpose(0, 2, 1).astype(bf)
    t_b2 = t_b2.reshape(E, -1, 1)
    t_w3 = (t_w3 * 0.5).transpose(0, 2, 1).astype(bf)
    t_b3 = t_b3.reshape(E, -1, 1)
    s_w1 = s_w1.transpose(0, 2, 1).astype(bf)
    s_b1 = s_b1.reshape(E, -1, 1)
    s_w2 = (s_w2 * 0.5).transpose(0, 2, 1).astype(bf)
    s_b2 = s_b2.reshape(E, -1, 1)
    s_w3 = s_w3 * 0.5

    TILE = 4096
    PERM = B // TILE
    grid = (N // TILE,)

    def tok_map(i):
        return (i, 0)

    def mmaj_map(i):
        return (i // PERM, i % PERM)

    def const_map2(i):
        return (0, 0)

    def const_map3(i):
        return (0, 0, 0)

    full2 = lambda a: pl.BlockSpec(a.shape, const_map2)
    full3 = lambda a: pl.BlockSpec(a.shape, const_map3)

    traj, score, probs, psum = pl.pallas_call(
        _fused_body,
        grid=grid,
        in_specs=[
            pl.BlockSpec((TILE, D), tok_map),
            full2(r_w1), full2(r_b1), full2(r_w2), full2(r_b2),
            full2(r_w3), full2(r_b3),
            full3(t_w1), full3(t_b1), full3(t_w2), full3(t_b2),
            full3(t_w3), full3(t_b3),
            full3(s_w1), full3(s_b1), full3(s_w2), full3(s_b2),
            full3(s_w3), full2(s_b3),
        ],
        out_specs=[
            pl.BlockSpec((OUT, TILE), mmaj_map),
            pl.BlockSpec((1, 1, TILE), lambda i: (i // PERM, 0, i % PERM)),
            pl.BlockSpec((1, E, TILE), lambda i: (i // PERM, 0, i % PERM)),
            pl.BlockSpec((1, 1, E), lambda i: (i, 0, 0)),
        ],
        out_shape=[
            jax.ShapeDtypeStruct((M * OUT, B), jnp.float32),
            jax.ShapeDtypeStruct((M, 1, B), jnp.float32),
            jax.ShapeDtypeStruct((M, E, B), jnp.float32),
            jax.ShapeDtypeStruct((grid[0], 1, E), jnp.float32),
        ],
    )(x, r_w1, r_b1, r_w2, r_b2, r_w3, r_b3,
      t_w1, t_b1, t_w2, t_b2, t_w3, t_b3,
      s_w1, s_b1, s_w2, s_b2, s_w3, s_b3)

    trajectories = traj.reshape(M, FUT, 2, B).transpose(3, 0, 1, 2)
    scores = score.reshape(M, B).transpose(1, 0)
    probs_out = probs.transpose(2, 0, 1)
    avg = psum.reshape(-1, E).sum(axis=0) / N
    entropy = -(avg * jnp.log(avg + 1e-08)).sum()
    load_balance_loss = -entropy * 0.01
    uniform = jnp.ones_like(avg) / E
    l2_loss = jnp.mean((avg - uniform) ** 2)
    aux_loss = load_balance_loss + 0.01 * l2_loss
    return (trajectories, scores, aux_loss, probs_out)


# speedup vs baseline: 5.3603x; 1.0337x over previous
import functools

import jax
import jax.numpy as jnp
from jax.experimental import pallas as pl

B, M, D = 4096, 6, 128
E, TOPK, FUT = 6, 2, 60
N = B * M
OUT = FUT * 2


def _gelu2(v):
    return v + v * jax.lax.erf(v * 0.7071067811865476)


def _fused_body(x_ref, r_w1, r_w2, r_b2, r_w3, r_b3,
                t_w1, t_w2, t_b2, t_w3, t_b3,
                s_w1, s_w2, s_b2, s_w3, s_b3,
                traj_ref, score_ref, probs_ref, psum_ref):
    f32 = jnp.float32
    dot = functools.partial(jnp.dot, preferred_element_type=f32)
    xt = jnp.transpose(x_ref[...])
    xt = jnp.concatenate([xt, jnp.ones((1, xt.shape[1]), f32)], axis=0)

    h = _gelu2(dot(r_w1[...], xt))
    h = _gelu2(dot(r_w2[...], h) + r_b2[...])
    logits = dot(r_w3[...], h) + r_b3[...]

    mx = jnp.max(logits, axis=0, keepdims=True)
    ex = jnp.exp(logits - mx)
    den = jnp.sum(ex, axis=0, keepdims=True)
    probs = ex / den
    T = xt.shape[1]
    probs_ref[...] = probs.reshape(1, E, T)
    psum_ref[...] = jnp.sum(probs, axis=1).reshape(1, 1, E)

    row = jax.lax.broadcasted_iota(jnp.int32, (E, T), 0)
    m0 = jnp.max(logits, axis=0)
    i0 = jnp.min(jnp.where(logits == m0[None, :], row, E), axis=0)
    masked = jnp.where(row == i0[None, :], -jnp.inf, logits)
    m1 = jnp.max(masked, axis=0)
    i1 = jnp.min(jnp.where(masked == m1[None, :], row, E), axis=0)
    e1 = jnp.exp(m1 - m0)
    p0 = 1.0 / (1.0 + e1)
    p1 = e1 / (1.0 + e1)
    wfull = (jnp.where(row == i0[None, :], p0[None, :], 0.0)
             + jnp.where(row == i1[None, :], p1[None, :], 0.0))

    bf = jnp.bfloat16
    xb = xt.astype(bf)
    acc_t = jnp.zeros((OUT, T), f32)
    acc_s = jnp.zeros((T,), f32)
    for e in range(E):
        th = _gelu2(dot(t_w1[e], xb))
        th = _gelu2(dot(t_w2[e], th.astype(bf)) + t_b2[e])
        tr = dot(t_w3[e], th.astype(bf)) + t_b3[e]
        sh = _gelu2(dot(s_w1[e], xb))
        sh = _gelu2(dot(s_w2[e], sh.astype(bf)) + s_b2[e])
        sc = jnp.sum(sh * s_w3[e], axis=0) + s_b3[e, 0]
        we = wfull[e:e + 1, :]
        acc_t = acc_t + we * tr
        acc_s = acc_s + wfull[e] * sc
    traj_ref[...] = acc_t
    score_ref[...] = acc_s.reshape(1, 1, T)


def kernel(mode_features, r_w1, r_b1, r_w2, r_b2, r_w3, r_b3,
           t_w1, t_b1, t_w2, t_b2, t_w3, t_b3,
           s_w1, s_b1, s_w2, s_b2, s_w3, s_b3):
    x = jnp.transpose(mode_features, (1, 0, 2)).reshape(N, D)
    bf = jnp.bfloat16
    r_w1 = jnp.concatenate([r_w1.T, r_b1.reshape(-1, 1)], axis=1)
    r_w2 = (r_w2 * 0.5).T
    r_b2 = r_b2.reshape(-1, 1)
    r_w3 = (r_w3 * 0.5).T
    r_b3 = r_b3.reshape(-1, 1)
    t_w1 = jnp.concatenate(
        [t_w1.transpose(0, 2, 1), t_b1.reshape(E, -1, 1)], axis=2).astype(bf)
    t_w2 = (t_w2 * 0.5).transpose(0, 2, 1).astype(bf)
    t_b2 = t_b2.reshape(E, -1, 1)
    t_w3 = (t_w3 * 0.5).transpose(0, 2, 1).astype(bf)
    t_b3 = t_b3.reshape(E, -1, 1)
    s_w1 = jnp.concatenate(
        [s_w1.transpose(0, 2, 1), s_b1.reshape(E, -1, 1)], axis=2).astype(bf)
    s_w2 = (s_w2 * 0.5).transpose(0, 2, 1).astype(bf)
    s_b2 = s_b2.reshape(E, -1, 1)
    s_w3 = s_w3 * 0.5

    TILE = 4096
    PERM = B // TILE
    grid = (N // TILE,)

    def tok_map(i):
        return (i, 0)

    def mmaj_map(i):
        return (i // PERM, i % PERM)

    def const_map2(i):
        return (0, 0)

    def const_map3(i):
        return (0, 0, 0)

    full2 = lambda a: pl.BlockSpec(a.shape, const_map2)
    full3 = lambda a: pl.BlockSpec(a.shape, const_map3)

    traj, score, probs, psum = pl.pallas_call(
        _fused_body,
        grid=grid,
        in_specs=[
            pl.BlockSpec((TILE, D), tok_map),
            full2(r_w1), full2(r_w2), full2(r_b2),
            full2(r_w3), full2(r_b3),
            full3(t_w1), full3(t_w2), full3(t_b2),
            full3(t_w3), full3(t_b3),
            full3(s_w1), full3(s_w2), full3(s_b2),
            full3(s_w3), full2(s_b3),
        ],
        out_specs=[
            pl.BlockSpec((OUT, TILE), mmaj_map),
            pl.BlockSpec((1, 1, TILE), lambda i: (i // PERM, 0, i % PERM)),
            pl.BlockSpec((1, E, TILE), lambda i: (i // PERM, 0, i % PERM)),
            pl.BlockSpec((1, 1, E), lambda i: (i, 0, 0)),
        ],
        out_shape=[
            jax.ShapeDtypeStruct((M * OUT, B), jnp.float32),
            jax.ShapeDtypeStruct((M, 1, B), jnp.float32),
            jax.ShapeDtypeStruct((M, E, B), jnp.float32),
            jax.ShapeDtypeStruct((grid[0], 1, E), jnp.float32),
        ],
    )(x, r_w1, r_w2, r_b2, r_w3, r_b3,
      t_w1, t_w2, t_b2, t_w3, t_b3,
      s_w1, s_w2, s_b2, s_w3, s_b3)

    trajectories = traj.reshape(M, FUT, 2, B).transpose(3, 0, 1, 2)
    scores = score.reshape(M, B).transpose(1, 0)
    probs_out = probs.transpose(2, 0, 1)
    avg = psum.reshape(-1, E).sum(axis=0) / N
    entropy = -(avg * jnp.log(avg + 1e-08)).sum()
    load_balance_loss = -entropy * 0.01
    uniform = jnp.ones_like(avg) / E
    l2_loss = jnp.mean((avg - uniform) ** 2)
    aux_loss = load_balance_loss + 0.01 * l2_loss
    return (trajectories, scores, aux_loss, probs_out)
